# Initial kernel scaffold; baseline (speedup 1.0000x reference)
import functools

import jax
import jax.numpy as jnp
from jax import lax
from jax.experimental import pallas as pl
from jax.experimental.pallas import tpu as pltpu
from jax.experimental.pallas import tpu_sc as plsc

_NUM_EMBED = 1000000
_EMBED_DIM = 32
_BATCH = 16384
_FIELDS = 26

_NC = 2
_NS = 16
_NW = _NC * _NS

_B = _BATCH * _FIELDS
_PER_W = _B // _NW
_CHUNK = 128
_CHUNKS = _PER_W // _CHUNK


def _gather_body(table_hbm, idx_hbm, out_hbm, idx_v, rows_v, sem):
    wid = lax.axis_index("s") * _NC + lax.axis_index("c")
    base = wid * _PER_W
    pltpu.sync_copy(idx_hbm.at[wid], idx_v)

    @pl.loop(0, _CHUNKS)
    def _(j):
        pltpu.async_copy(table_hbm.at[idx_v.at[j]], rows_v, sem).wait()
        pltpu.sync_copy(rows_v, out_hbm.at[pl.ds(base + j * _CHUNK, _CHUNK)])


@jax.jit
def _gather(table, idx):
    mesh = plsc.VectorSubcoreMesh(core_axis_name="c", subcore_axis_name="s")
    return pl.kernel(
        _gather_body,
        out_type=jax.ShapeDtypeStruct((_B, _EMBED_DIM), jnp.float32),
        mesh=mesh,
        scratch_types=[
            pltpu.VMEM((_CHUNKS, _CHUNK), jnp.int32),
            pltpu.VMEM((_CHUNK, _EMBED_DIM), jnp.float32),
            pltpu.SemaphoreType.DMA,
        ],
    )(table, idx)


def kernel(x, table):
    idx = x.astype(jnp.int32).reshape(_NW, _CHUNKS, _CHUNK)
    out = _gather(table, idx)
    return out.reshape(_BATCH, _FIELDS, _EMBED_DIM)


# speedup vs baseline: 1.4378x; 1.4378x over previous
import functools

import jax
import jax.numpy as jnp
from jax import lax
from jax.experimental import pallas as pl
from jax.experimental.pallas import tpu as pltpu
from jax.experimental.pallas import tpu_sc as plsc

_NUM_EMBED = 1000000
_EMBED_DIM = 32
_BATCH = 16384
_FIELDS = 26

_NC = 2
_NS = 16
_NW = _NC * _NS

_B = _BATCH * _FIELDS
_PER_W = _B // _NW
_CHUNK = 128
_CHUNKS = _PER_W // _CHUNK


def _gather_body(table_hbm, idx_hbm, out_hbm, idx_v, rows_v, sem):
    wid = lax.axis_index("s") * _NC + lax.axis_index("c")
    base = wid * _PER_W
    pltpu.sync_copy(idx_hbm.at[wid], idx_v)

    @pl.loop(0, _CHUNKS)
    def _(j):
        pltpu.async_copy(table_hbm.at[idx_v.at[j]], rows_v, sem).wait()
        pltpu.sync_copy(rows_v, out_hbm.at[pl.ds(base + j * _CHUNK, _CHUNK)])


@jax.jit
def _gather(table, idx):
    mesh = plsc.VectorSubcoreMesh(core_axis_name="c", subcore_axis_name="s")
    return pl.kernel(
        _gather_body,
        out_type=jax.ShapeDtypeStruct((_B, _EMBED_DIM), jnp.float32),
        mesh=mesh,
        scratch_types=[
            pltpu.VMEM((_CHUNKS, _CHUNK), jnp.int32),
            pltpu.VMEM((_CHUNK, _EMBED_DIM), jnp.float32),
            pltpu.SemaphoreType.DMA,
        ],
        compiler_params=pltpu.CompilerParams(use_tc_tiling_on_sc=False),
    )(table, idx)


def kernel(x, table):
    idx = x.astype(jnp.int32).reshape(_NW, _CHUNKS, _CHUNK)
    out = _gather(table, idx)
    return out.reshape(_BATCH, _FIELDS, _EMBED_DIM)


# speedup vs baseline: 1.5619x; 1.0863x over previous
import jax
import jax.numpy as jnp
from jax import lax
from jax.experimental import pallas as pl
from jax.experimental.pallas import tpu as pltpu
from jax.experimental.pallas import tpu_sc as plsc

_NUM_EMBED = 1000000
_EMBED_DIM = 32
_BATCH = 16384
_FIELDS = 26

_NC = 2
_NS = 16
_NW = _NC * _NS

_B = _BATCH * _FIELDS
_PER_W = _B // _NW
_CHUNK = 128
_CHUNKS = _PER_W // _CHUNK
_SEG_CHUNKS = 4
_SEG_ROWS = _SEG_CHUNKS * _CHUNK
_NSEG = _CHUNKS // _SEG_CHUNKS


def _gather_body(table_hbm, idx_hbm, out_hbm, idx_v, rows0, rows1,
                 g0, g1, w0, w1):
    wid = lax.axis_index("s") * _NC + lax.axis_index("c")
    base = wid * _PER_W
    pltpu.sync_copy(idx_hbm.at[wid], idx_v)

    def fire(seg, rows, gsem):
        for k in range(_SEG_CHUNKS):
            pltpu.async_copy(
                table_hbm.at[idx_v.at[seg * _SEG_CHUNKS + k]],
                rows.at[pl.ds(k * _CHUNK, _CHUNK)], gsem)

    def drain_gather(rows, gsem):
        pltpu.make_async_copy(
            table_hbm.at[pl.ds(0, _SEG_ROWS)], rows, gsem).wait()

    def write(seg, rows, wsem):
        pltpu.async_copy(
            rows, out_hbm.at[pl.ds(base + seg * _SEG_ROWS, _SEG_ROWS)], wsem)

    def drain_write(rows, wsem):
        pltpu.make_async_copy(
            rows, out_hbm.at[pl.ds(base, _SEG_ROWS)], wsem).wait()

    fire(0, rows0, g0)
    fire(1, rows1, g1)

    @pl.loop(0, _NSEG // 2 - 1)
    def _(t):
        s0 = 2 * t
        drain_gather(rows0, g0)
        write(s0, rows0, w0)
        drain_gather(rows1, g1)
        write(s0 + 1, rows1, w1)
        drain_write(rows0, w0)
        fire(s0 + 2, rows0, g0)
        drain_write(rows1, w1)
        fire(s0 + 3, rows1, g1)

    drain_gather(rows0, g0)
    write(_NSEG - 2, rows0, w0)
    drain_gather(rows1, g1)
    write(_NSEG - 1, rows1, w1)
    drain_write(rows0, w0)
    drain_write(rows1, w1)


@jax.jit
def _gather(table, idx):
    mesh = plsc.VectorSubcoreMesh(core_axis_name="c", subcore_axis_name="s")
    return pl.kernel(
        _gather_body,
        out_type=jax.ShapeDtypeStruct((_B, _EMBED_DIM), jnp.float32),
        mesh=mesh,
        scratch_types=[
            pltpu.VMEM((_CHUNKS, _CHUNK), jnp.int32),
            pltpu.VMEM((_SEG_ROWS, _EMBED_DIM), jnp.float32),
            pltpu.VMEM((_SEG_ROWS, _EMBED_DIM), jnp.float32),
            pltpu.SemaphoreType.DMA,
            pltpu.SemaphoreType.DMA,
            pltpu.SemaphoreType.DMA,
            pltpu.SemaphoreType.DMA,
        ],
        compiler_params=pltpu.CompilerParams(use_tc_tiling_on_sc=False),
    )(table, idx)


def kernel(x, table):
    idx = x.astype(jnp.int32).reshape(_NW, _CHUNKS, _CHUNK)
    out = _gather(table, idx)
    return out.reshape(_BATCH, _FIELDS, _EMBED_DIM)


# speedup vs baseline: 1.5726x; 1.0068x over previous
import jax
import jax.numpy as jnp
from jax import lax
from jax.experimental import pallas as pl
from jax.experimental.pallas import tpu as pltpu
from jax.experimental.pallas import tpu_sc as plsc

_NUM_EMBED = 1000000
_EMBED_DIM = 32
_BATCH = 16384
_FIELDS = 26

_NC = 2
_NS = 16
_NW = _NC * _NS

_B = _BATCH * _FIELDS
_PER_W = _B // _NW
_CHUNK = 128
_CHUNKS = _PER_W // _CHUNK
_SEG_CHUNKS = 2
_SEG_ROWS = _SEG_CHUNKS * _CHUNK
_NSEG = _CHUNKS // _SEG_CHUNKS
_NBUF = 4
_ROUNDS = _NSEG // _NBUF


def _gather_body(table_hbm, idx_hbm, out_hbm, idx_v, *rest):
    rows = rest[:_NBUF]
    gsem = rest[_NBUF:2 * _NBUF]
    wsem = rest[2 * _NBUF:3 * _NBUF]

    wid = lax.axis_index("s") * _NC + lax.axis_index("c")
    base = wid * _PER_W
    pltpu.sync_copy(idx_hbm.at[wid], idx_v)

    def fire(seg, b):
        for k in range(_SEG_CHUNKS):
            pltpu.async_copy(
                table_hbm.at[idx_v.at[seg * _SEG_CHUNKS + k]],
                rows[b].at[pl.ds(k * _CHUNK, _CHUNK)], gsem[b])

    def drain_gather(b):
        pltpu.make_async_copy(
            table_hbm.at[pl.ds(0, _SEG_ROWS)], rows[b], gsem[b]).wait()

    def write(seg, b):
        pltpu.async_copy(
            rows[b], out_hbm.at[pl.ds(base + seg * _SEG_ROWS, _SEG_ROWS)],
            wsem[b])

    def drain_write(b):
        pltpu.make_async_copy(
            rows[b], out_hbm.at[pl.ds(base, _SEG_ROWS)], wsem[b]).wait()

    for b in range(_NBUF):
        fire(b, b)

    @pl.loop(0, _ROUNDS - 1)
    def _(t):
        s0 = t * _NBUF
        for b in range(_NBUF):
            drain_gather(b)
            write(s0 + b, b)
        for b in range(_NBUF):
            drain_write(b)
            fire(s0 + _NBUF + b, b)

    s0 = (_ROUNDS - 1) * _NBUF
    for b in range(_NBUF):
        drain_gather(b)
        write(s0 + b, b)
    for b in range(_NBUF):
        drain_write(b)


@jax.jit
def _gather(table, idx):
    mesh = plsc.VectorSubcoreMesh(core_axis_name="c", subcore_axis_name="s")
    return pl.kernel(
        _gather_body,
        out_type=jax.ShapeDtypeStruct((_B, _EMBED_DIM), jnp.float32),
        mesh=mesh,
        scratch_types=(
            [pltpu.VMEM((_CHUNKS, _CHUNK), jnp.int32)]
            + [pltpu.VMEM((_SEG_ROWS, _EMBED_DIM), jnp.float32)] * _NBUF
            + [pltpu.SemaphoreType.DMA] * (2 * _NBUF)
        ),
        compiler_params=pltpu.CompilerParams(use_tc_tiling_on_sc=False),
    )(table, idx)


def kernel(x, table):
    idx = x.astype(jnp.int32).reshape(_NW, _CHUNKS, _CHUNK)
    out = _gather(table, idx)
    return out.reshape(_BATCH, _FIELDS, _EMBED_DIM)
